# baseline (speedup 1.0000x reference)
import functools

import jax
import jax.numpy as jnp
from jax import lax
from jax.experimental import pallas as pl
from jax.experimental.pallas import tpu as pltpu
from jax.experimental.pallas import tpu_sc as plsc

_GAMMA = 12.0
_B = 16384
_DIM = 64
_NC = 1
_NS = 16
_NW = _NC * _NS
_BPW = _B // _NW
_NCHUNK = 8
_CHUNK = _BPW // _NCHUNK
_GPC = _CHUNK // 16
_VROWS = 500000


def _make_kernel():
    mesh = plsc.VectorSubcoreMesh(
        core_axis_name="c", subcore_axis_name="s",
        num_cores=_NC, num_subcores=_NS,
    )

    @functools.partial(
        pl.kernel,
        out_type=jax.ShapeDtypeStruct((_NW, _BPW), jnp.float32),
        mesh=mesh,
        compiler_params=pltpu.CompilerParams(needs_layout_passes=False),
        scratch_types=[
            pltpu.VMEM((_BPW * 3,), jnp.int32),
            pltpu.VMEM((_NCHUNK, _CHUNK), jnp.int32),
            pltpu.VMEM((_NCHUNK, _CHUNK), jnp.int32),
            pltpu.VMEM((_NCHUNK, _CHUNK), jnp.int32),
            pltpu.VMEM((_BPW,), jnp.int32),
            pltpu.VMEM((2, _CHUNK, 128), jnp.float32),
            pltpu.VMEM((2, _CHUNK, 128), jnp.float32),
            pltpu.VMEM((2, _CHUNK, 128), jnp.float32),
            pltpu.VMEM((16, 17), jnp.float32),
            pltpu.VMEM((_BPW,), jnp.float32),
            pltpu.SemaphoreType.DMA,
        ],
    )
    def kge_score(samp_hbm, ent_hbm, rel_hbm, out_hbm,
                  samp_v, hidx, ridx, tidx, pofs,
                  hrow, rrow, trow, wtile, out_v, sem):
        wid = lax.axis_index("s") * _NC + lax.axis_index("c")

        pltpu.sync_copy(samp_hbm.at[wid], samp_v)

        lanes = lax.iota(jnp.int32, 16)
        col_dst = (hidx, ridx, tidx)
        for g in range(_BPW // 16):
            j, r0 = divmod(g * 16, _CHUNK)
            packed = jnp.zeros((16,), jnp.int32)
            for c in range(3):
                v = plsc.load_gather(samp_v, [lanes * 3 + (g * 48 + c)])
                col_dst[c][j, pl.ds(r0, 16)] = v >> 1
                packed = packed | ((v & 1) << (6 + 8 * c))
            pofs[pl.ds(g * 16, 16)] = packed

        def fire(k, slot):
            return [
                pltpu.async_copy(ent_hbm.at[hidx.at[k]], hrow.at[slot], sem),
                pltpu.async_copy(rel_hbm.at[ridx.at[k]], rrow.at[slot], sem),
                pltpu.async_copy(ent_hbm.at[tidx.at[k]], trow.at[slot], sem),
            ]

        def score_chunk(k, slot):
            def group_body(g, carry):
                s0 = k * _CHUNK + g * 16
                pvec = pofs[pl.ds(s0, 16)]
                for row in range(16):
                    r = g * 16 + row
                    w = pvec[row]
                    ho = pl.multiple_of(w & 64, 16)
                    ro = pl.multiple_of((w >> 8) & 64, 16)
                    to = pl.multiple_of((w >> 16) & 64, 16)
                    acc = jnp.zeros((16,), jnp.float32)
                    for c in range(_DIM // 16):
                        hv = hrow[slot, r, pl.ds(ho + c * 16, 16)]
                        rv = rrow[slot, r, pl.ds(ro + c * 16, 16)]
                        tv = trow[slot, r, pl.ds(to + c * 16, 16)]
                        acc = acc + jnp.abs(hv + rv - tv)
                    wtile[row, pl.ds(0, 16)] = acc
                tot = jnp.zeros((16,), jnp.float32)
                for d in range(16):
                    tot = tot + plsc.load_gather(
                        wtile, [lanes, jnp.full((16,), d, jnp.int32)])
                out_v[pl.ds(s0, 16)] = _GAMMA - tot
                return carry
            lax.fori_loop(0, _GPC, group_body, 0)

        pending = fire(0, 0)
        for k in range(_NCHUNK):
            for cp in pending:
                cp.wait()
            if k + 1 < _NCHUNK:
                pending = fire(k + 1, (k + 1) % 2)
            score_chunk(k, k % 2)

        pltpu.sync_copy(out_v, out_hbm.at[wid])

    return kge_score


_kge_score = _make_kernel()


def kernel(sample, entity_embedding, relation_embedding):
    samp = sample.astype(jnp.int32).reshape(_NW, _BPW * 3)
    ent2 = entity_embedding.reshape(_VROWS, 2 * _DIM)
    rel2 = relation_embedding.reshape(_VROWS, 2 * _DIM)
    out = _kge_score(samp, ent2, rel2)
    return out.reshape(_B, 1)


# speedup vs baseline: 2.0641x; 2.0641x over previous
import functools

import jax
import jax.numpy as jnp
from jax import lax
from jax.experimental import pallas as pl
from jax.experimental.pallas import tpu as pltpu
from jax.experimental.pallas import tpu_sc as plsc

_GAMMA = 12.0
_B = 16384
_DIM = 64
_NC = 2
_NS = 16
_NW = _NC * _NS
_BPW = _B // _NW
_NGRP = _BPW // 16
_TROWS = 125000


def _make_kernel():
    mesh = plsc.VectorSubcoreMesh(
        core_axis_name="c", subcore_axis_name="s",
        num_cores=_NC, num_subcores=_NS,
    )

    @functools.partial(
        pl.kernel,
        out_type=jax.ShapeDtypeStruct((_NW, _BPW), jnp.float32),
        mesh=mesh,
        compiler_params=pltpu.CompilerParams(needs_layout_passes=False),
        scratch_types=[
            pltpu.VMEM((_BPW * 3,), jnp.int32),
            pltpu.VMEM((_BPW,), jnp.int32),
            pltpu.VMEM((_BPW,), jnp.int32),
            pltpu.VMEM((_BPW,), jnp.int32),
            pltpu.VMEM((_BPW,), jnp.int32),
            pltpu.VMEM((16, 8, _DIM), jnp.float32),
            pltpu.VMEM((16, 8, _DIM), jnp.float32),
            pltpu.VMEM((16, 8, _DIM), jnp.float32),
            pltpu.VMEM((16, 17), jnp.float32),
            pltpu.VMEM((_BPW,), jnp.float32),
            pltpu.SemaphoreType.DMA,
        ],
    )
    def kge_score(samp_hbm, ent_hbm, rel_hbm, out_hbm,
                  samp_v, hidx, ridx, tidx, pofs,
                  hblk, rblk, tblk, wtile, out_v, sem):
        wid = lax.axis_index("s") * _NC + lax.axis_index("c")

        pltpu.sync_copy(samp_hbm.at[wid], samp_v)

        lanes = lax.iota(jnp.int32, 16)
        col_dst = (hidx, ridx, tidx)
        for g in range(_NGRP):
            packed = jnp.zeros((16,), jnp.int32)
            for c in range(3):
                v = plsc.load_gather(samp_v, [lanes * 3 + (g * 48 + c)])
                col_dst[c][pl.ds(g * 16, 16)] = v >> 3
                packed = packed | ((v & 7) << (8 * c))
            pofs[pl.ds(g * 16, 16)] = packed

        def group_body(g, carry):
            s0 = g * 16
            hvec = hidx[pl.ds(s0, 16)]
            rvec = ridx[pl.ds(s0, 16)]
            tvec = tidx[pl.ds(s0, 16)]
            copies = []
            for row in range(16):
                copies.append(
                    pltpu.async_copy(ent_hbm.at[hvec[row]], hblk.at[row], sem))
                copies.append(
                    pltpu.async_copy(rel_hbm.at[rvec[row]], rblk.at[row], sem))
                copies.append(
                    pltpu.async_copy(ent_hbm.at[tvec[row]], tblk.at[row], sem))
            for cp in copies:
                cp.wait()

            pvec = pofs[pl.ds(s0, 16)]
            for row in range(16):
                w = pvec[row]
                hs = w & 7
                rs = (w >> 8) & 7
                ts = (w >> 16) & 7
                acc = jnp.zeros((16,), jnp.float32)
                for c in range(_DIM // 16):
                    hv = hblk[row, hs, pl.ds(c * 16, 16)]
                    rv = rblk[row, rs, pl.ds(c * 16, 16)]
                    tv = tblk[row, ts, pl.ds(c * 16, 16)]
                    acc = acc + jnp.abs(hv + rv - tv)
                wtile[row, pl.ds(0, 16)] = acc
            tot = jnp.zeros((16,), jnp.float32)
            for d in range(16):
                tot = tot + plsc.load_gather(
                    wtile, [lanes, jnp.full((16,), d, jnp.int32)])
            out_v[pl.ds(s0, 16)] = _GAMMA - tot
            return carry

        lax.fori_loop(0, _NGRP, group_body, 0)

        pltpu.sync_copy(out_v, out_hbm.at[wid])

    return kge_score


_kge_score = _make_kernel()


def kernel(sample, entity_embedding, relation_embedding):
    samp = sample.astype(jnp.int32).reshape(_NW, _BPW * 3)
    ent3 = entity_embedding.reshape(_TROWS, 8, _DIM)
    rel3 = relation_embedding.reshape(_TROWS, 8, _DIM)
    out = _kge_score(samp, ent3, rel3)
    return out.reshape(_B, 1)
